# baseline (speedup 1.0000x reference)
import functools

import jax
import jax.numpy as jnp
from jax import lax
from jax.experimental import pallas as pl
from jax.experimental.pallas import tpu as pltpu
from jax.experimental.pallas import tpu_sc as plsc

N = 10000
D = 128
G = 64
NP = 10240
NC = 2
NS = 16
NW = NC * NS
EK = 128
NCH0 = 105
RPT = NP // NS
RCH = RPT // EK
NB = 10
BN = NP // NB


def _deg_body(dst_hbm, w_hbm, deg_out, dst_v, w_v, t640_v, dacc, lsem):
    cid = lax.axis_index("c")
    sid = lax.axis_index("s")
    wid = sid * NC + cid
    ept = dst_hbm.shape[0] // NW
    nch = ept // EK

    def zbody(i, c):
        t640_v[pl.ds(i * 16, 16)] = jnp.zeros((16,), jnp.float32)
        return c
    lax.fori_loop(0, RPT // 16, zbody, 0)
    pltpu.sync_copy(t640_v, dacc.at[pl.ds(sid * RPT, RPT)])
    plsc.subcore_barrier()

    def load(b, g):
        base = wid * ept + g * EK
        pltpu.async_copy(dst_hbm.at[pl.ds(base, EK)], dst_v[b], lsem[b])
        pltpu.async_copy(w_hbm.at[pl.ds(base, EK)], w_v[b], lsem[b])

    def wait_load(b, g):
        base = wid * ept + g * EK
        pltpu.make_async_copy(
            dst_hbm.at[pl.ds(base, EK)], dst_v[b], lsem[b]).wait()
        pltpu.make_async_copy(
            w_hbm.at[pl.ds(base, EK)], w_v[b], lsem[b]).wait()

    def dproc(b, g, prefetch_other, g_other):
        @pl.when(prefetch_other)
        def _():
            load(1 - b, g_other)
        wait_load(b, g)
        pltpu.sync_copy(w_v[b], dacc.at[dst_v[b]], add=True)

    load(0, 0)

    def pair(i, c):
        g0 = 2 * i
        dproc(0, g0, True, g0 + 1)
        dproc(1, g0 + 1, g0 + 2 < nch, g0 + 2)
        return c
    lax.fori_loop(0, nch // 2, pair, 0)
    dproc(0, nch - 1, False, 0)
    plsc.subcore_barrier()

    pltpu.sync_copy(dacc.at[pl.ds(sid * RPT, RPT)], t640_v)
    pltpu.sync_copy(t640_v, deg_out.at[cid, pl.ds(sid * RPT, RPT)])


def _spmm_body(h_hbm, src_hbm, dst_hbm, w_hbm, s_out,
               src_v, dst_v, w_v, rows_v, gsem, ssem, acc):
    cid = lax.axis_index("c")
    sid = lax.axis_index("s")
    npair = (src_hbm.shape[0] // EK) // NS
    nch = jnp.where(cid == 0, NCH0, npair - NCH0)
    coff = jnp.where(cid == 0, 0, NCH0)
    cbase = sid * npair + coff

    with jax.named_scope("spmm_init"):
        def zrow(e, c):
            for k in range(D // 16):
                rows_v[0][e, pl.ds(k * 16, 16)] = jnp.zeros((16,),
                                                            jnp.float32)
            return c
        lax.fori_loop(0, EK, zrow, 0)
        for j in range(RCH):
            pltpu.sync_copy(rows_v[0], acc.at[pl.ds(sid * RPT + j * EK, EK)])
        plsc.subcore_barrier()

    def load_src(b, g):
        pltpu.sync_copy(src_hbm.at[pl.ds((cbase + g) * EK, EK)], src_v[b])

    def issue_gather(b):
        return pltpu.async_copy(h_hbm.at[src_v[b]], rows_v[b], gsem[b])

    def wait_gather(b):
        pltpu.make_async_copy(h_hbm.at[src_v[b]], rows_v[b], gsem[b]).wait()

    def issue_scatter(b):
        pltpu.async_copy(rows_v[b], acc.at[dst_v[b]], ssem[b], add=True)

    def wait_scatter(b):
        pltpu.make_async_copy(rows_v[b], acc.at[dst_v[b]], ssem[b]).wait()

    def scale(b):
        def srow(g16, cc):
            w16 = w_v[b][pl.ds(g16 * 16, 16)]
            for j in range(16):
                e = g16 * 16 + j
                ws = w16[j]
                for k in range(D // 16):
                    sl = pl.ds(k * 16, 16)
                    rows_v[b][e, sl] = rows_v[b][e, sl] * ws
            return cc
        lax.fori_loop(0, EK // 16, srow, 0, unroll=4)

    def process(b, g, prefetch_other, g_other):
        @pl.when(prefetch_other)
        def _():
            load_src(1 - b, g_other)
            issue_gather(1 - b)
        @pl.when(g >= 2)
        def _():
            wait_scatter(b)
        pltpu.sync_copy(dst_hbm.at[pl.ds((cbase + g) * EK, EK)], dst_v[b])
        pltpu.sync_copy(w_hbm.at[pl.ds((cbase + g) * EK, EK)], w_v[b])
        wait_gather(b)
        scale(b)
        issue_scatter(b)

    with jax.named_scope("spmm_edges"):
        load_src(0, 0)
        issue_gather(0)

        def pair(i, c):
            g0 = 2 * i
            process(0, g0, True, g0 + 1)
            process(1, g0 + 1, g0 + 2 < nch, g0 + 2)
            return c
        lax.fori_loop(0, nch // 2, pair, 0)
        process(0, nch - 1, False, 0)
        wait_scatter(1)
        wait_scatter(0)
        plsc.subcore_barrier()

    with jax.named_scope("spmm_writeout"):
        for j in range(RCH):
            r0 = sid * RPT + j * EK
            pltpu.sync_copy(acc.at[pl.ds(r0, EK)], rows_v[0])
            pltpu.sync_copy(rows_v[0], s_out.at[cid, pl.ds(r0, EK)])


_SC_MESH = plsc.VectorSubcoreMesh(core_axis_name="c", subcore_axis_name="s")

_deg_call = pl.kernel(
    _deg_body,
    out_type=jax.ShapeDtypeStruct((NC, NP), jnp.float32),
    mesh=_SC_MESH,
    scratch_types=[
        (pltpu.VMEM((EK,), jnp.int32), pltpu.VMEM((EK,), jnp.int32)),
        (pltpu.VMEM((EK,), jnp.float32), pltpu.VMEM((EK,), jnp.float32)),
        pltpu.VMEM((RPT,), jnp.float32),
        pltpu.VMEM_SHARED((NP,), jnp.float32),
        (pltpu.SemaphoreType.DMA, pltpu.SemaphoreType.DMA),
    ],
)

_spmm_call = pl.kernel(
    _spmm_body,
    out_type=jax.ShapeDtypeStruct((NC, NP, D), jnp.float32),
    mesh=_SC_MESH,
    scratch_types=[
        (pltpu.VMEM((EK,), jnp.int32), pltpu.VMEM((EK,), jnp.int32)),
        (pltpu.VMEM((EK,), jnp.int32), pltpu.VMEM((EK,), jnp.int32)),
        (pltpu.VMEM((EK,), jnp.float32), pltpu.VMEM((EK,), jnp.float32)),
        (pltpu.VMEM((EK, D), jnp.float32), pltpu.VMEM((EK, D), jnp.float32)),
        (pltpu.SemaphoreType.DMA, pltpu.SemaphoreType.DMA),
        (pltpu.SemaphoreType.DMA, pltpu.SemaphoreType.DMA),
        pltpu.VMEM_SHARED((NP, D), jnp.float32),
    ],
)


def _tc1_body(x_ref, w1_ref, deg_ref, h1p_ref, dis_ref):
    degs = deg_ref[0] + deg_ref[1] + 1.0
    s = jnp.where(degs > 0, lax.rsqrt(jnp.where(degs > 0, degs, 1.0)), 0.0)
    h = jnp.dot(x_ref[...], w1_ref[...], preferred_element_type=jnp.float32)
    h1p_ref[...] = h * s
    dis_ref[...] = s


def _tc2_body(s1_ref, h1p_ref, dis_ref, b1_ref, w2_ref, h2p_ref):
    s = dis_ref[...]
    pre = (s1_ref[0] + s1_ref[1] + h1p_ref[...]) * s + b1_ref[...]
    o = jnp.maximum(pre, 0.0)
    h2 = jnp.dot(o, w2_ref[...], preferred_element_type=jnp.float32)
    h2p_ref[...] = h2 * s


def _tc3_body(s2_ref, h2p_ref, dis_ref, b2_ref, batch_ref, out_ref):
    i = pl.program_id(0)
    s = dis_ref[...]
    pre = (s2_ref[0] + s2_ref[1] + h2p_ref[...]) * s + b2_ref[...]
    o = jnp.maximum(pre, 0.0)
    seg = lax.broadcasted_iota(jnp.int32, (BN, G), 1)
    oh = (batch_ref[...] == seg).astype(jnp.float32)
    contrib = lax.dot_general(oh, o, (((0,), (0,)), ((), ())),
                              preferred_element_type=jnp.float32)

    @pl.when(i == 0)
    def _init():
        out_ref[...] = contrib

    @pl.when(i > 0)
    def _acc():
        out_ref[...] += contrib


_tc1_call = pl.pallas_call(
    _tc1_body,
    grid=(NB,),
    in_specs=[
        pl.BlockSpec((BN, D), lambda i: (i, 0)),
        pl.BlockSpec((D, D), lambda i: (0, 0)),
        pl.BlockSpec((NC, BN, 1), lambda i: (0, i, 0)),
    ],
    out_specs=[
        pl.BlockSpec((BN, D), lambda i: (i, 0)),
        pl.BlockSpec((BN, 1), lambda i: (i, 0)),
    ],
    out_shape=[
        jax.ShapeDtypeStruct((NP, D), jnp.float32),
        jax.ShapeDtypeStruct((NP, 1), jnp.float32),
    ],
)

_tc2_call = pl.pallas_call(
    _tc2_body,
    grid=(NB,),
    in_specs=[
        pl.BlockSpec((NC, BN, D), lambda i: (0, i, 0)),
        pl.BlockSpec((BN, D), lambda i: (i, 0)),
        pl.BlockSpec((BN, 1), lambda i: (i, 0)),
        pl.BlockSpec((1, D), lambda i: (0, 0)),
        pl.BlockSpec((D, D), lambda i: (0, 0)),
    ],
    out_specs=pl.BlockSpec((BN, D), lambda i: (i, 0)),
    out_shape=jax.ShapeDtypeStruct((NP, D), jnp.float32),
)

_tc3_call = pl.pallas_call(
    _tc3_body,
    grid=(NB,),
    in_specs=[
        pl.BlockSpec((NC, BN, D), lambda i: (0, i, 0)),
        pl.BlockSpec((BN, D), lambda i: (i, 0)),
        pl.BlockSpec((BN, 1), lambda i: (i, 0)),
        pl.BlockSpec((1, D), lambda i: (0, 0)),
        pl.BlockSpec((BN, 1), lambda i: (i, 0)),
    ],
    out_specs=pl.BlockSpec((G, D), lambda i: (0, 0)),
    out_shape=jax.ShapeDtypeStruct((G, D), jnp.float32),
)


@jax.jit
def kernel(x, edge_index, edge_weight, batch, W1, b1, W2, b2):
    e = edge_weight.shape[0]
    ept = -(-e // (NW * EK)) * EK
    epad = ept * NW
    pe = epad - e

    src = edge_index[0].astype(jnp.int32)
    dst = edge_index[1].astype(jnp.int32)
    src_p = jnp.concatenate([src, jnp.zeros((pe,), jnp.int32)])
    dst_p = jnp.concatenate([dst, jnp.zeros((pe,), jnp.int32)])
    w_p = jnp.concatenate([edge_weight, jnp.zeros((pe,), edge_weight.dtype)])

    x_p = jnp.concatenate([x, jnp.zeros((NP - N, D), x.dtype)])
    batch_p = jnp.concatenate(
        [batch.astype(jnp.int32), jnp.full((NP - N,), G, jnp.int32)]
    ).reshape(NP, 1)

    deg2 = _deg_call(dst_p, w_p).reshape(NC, NP, 1)
    h1p, dis = _tc1_call(x_p, W1, deg2)
    s1 = _spmm_call(h1p, src_p, dst_p, w_p)
    h2p = _tc2_call(s1, h1p, dis, b1.reshape(1, D), W2)
    s2 = _spmm_call(h2p, src_p, dst_p, w_p)
    return _tc3_call(s2, h2p, dis, b2.reshape(1, D), batch_p)


# speedup vs baseline: 1.0338x; 1.0338x over previous
import functools

import jax
import jax.numpy as jnp
from jax import lax
from jax.experimental import pallas as pl
from jax.experimental.pallas import tpu as pltpu
from jax.experimental.pallas import tpu_sc as plsc

N = 10000
D = 128
G = 64
NP = 10240
NC = 2
NS = 16
NW = NC * NS
EK = 128
RPT = NP // NS
RCH = RPT // EK
NB = 10
BN = NP // NB


def _deg_body(dst_hbm, w_hbm, deg_out, dst_v, w_v, t640_v, dacc, lsem):
    cid = lax.axis_index("c")
    sid = lax.axis_index("s")
    wid = sid * NC + cid
    ept = dst_hbm.shape[0] // NW
    nch = ept // EK

    def zbody(i, c):
        t640_v[pl.ds(i * 16, 16)] = jnp.zeros((16,), jnp.float32)
        return c
    lax.fori_loop(0, RPT // 16, zbody, 0)
    pltpu.sync_copy(t640_v, dacc.at[pl.ds(sid * RPT, RPT)])
    plsc.subcore_barrier()

    def load(b, g):
        base = wid * ept + g * EK
        pltpu.async_copy(dst_hbm.at[pl.ds(base, EK)], dst_v[b], lsem[b])
        pltpu.async_copy(w_hbm.at[pl.ds(base, EK)], w_v[b], lsem[b])

    def wait_load(b, g):
        base = wid * ept + g * EK
        pltpu.make_async_copy(
            dst_hbm.at[pl.ds(base, EK)], dst_v[b], lsem[b]).wait()
        pltpu.make_async_copy(
            w_hbm.at[pl.ds(base, EK)], w_v[b], lsem[b]).wait()

    def dproc(b, g, prefetch_other, g_other):
        @pl.when(prefetch_other)
        def _():
            load(1 - b, g_other)
        wait_load(b, g)
        pltpu.sync_copy(w_v[b], dacc.at[dst_v[b]], add=True)

    load(0, 0)

    def pair(i, c):
        g0 = 2 * i
        dproc(0, g0, True, g0 + 1)
        dproc(1, g0 + 1, g0 + 2 < nch, g0 + 2)
        return c
    lax.fori_loop(0, nch // 2, pair, 0)
    dproc(0, nch - 1, False, 0)
    plsc.subcore_barrier()

    pltpu.sync_copy(dacc.at[pl.ds(sid * RPT, RPT)], t640_v)
    pltpu.sync_copy(t640_v, deg_out.at[cid, pl.ds(sid * RPT, RPT)])


def _spmm_body(nch0, h_hbm, src_hbm, dst_hbm, w_hbm, s_out,
               src_v, dst_v, w_v, rows_v, gsem, ssem, acc):
    cid = lax.axis_index("c")
    sid = lax.axis_index("s")
    npair = (src_hbm.shape[0] // EK) // NS
    nch = jnp.where(cid == 0, nch0, npair - nch0)
    coff = jnp.where(cid == 0, 0, nch0)
    cbase = sid * npair + coff

    with jax.named_scope("spmm_init"):
        def zrow(e, c):
            for k in range(D // 16):
                rows_v[0][e, pl.ds(k * 16, 16)] = jnp.zeros((16,),
                                                            jnp.float32)
            return c
        lax.fori_loop(0, EK, zrow, 0)
        for j in range(RCH):
            pltpu.sync_copy(rows_v[0], acc.at[pl.ds(sid * RPT + j * EK, EK)])
        plsc.subcore_barrier()

    def load_src(b, g):
        pltpu.sync_copy(src_hbm.at[pl.ds((cbase + g) * EK, EK)], src_v[b])

    def issue_gather(b):
        return pltpu.async_copy(h_hbm.at[src_v[b]], rows_v[b], gsem[b])

    def wait_gather(b):
        pltpu.make_async_copy(h_hbm.at[src_v[b]], rows_v[b], gsem[b]).wait()

    def issue_scatter(b):
        pltpu.async_copy(rows_v[b], acc.at[dst_v[b]], ssem[b], add=True)

    def wait_scatter(b):
        pltpu.make_async_copy(rows_v[b], acc.at[dst_v[b]], ssem[b]).wait()

    def scale(b):
        def srow(g16, cc):
            w16 = w_v[b][pl.ds(g16 * 16, 16)]
            for j in range(16):
                e = g16 * 16 + j
                ws = w16[j]
                for k in range(D // 16):
                    sl = pl.ds(k * 16, 16)
                    rows_v[b][e, sl] = rows_v[b][e, sl] * ws
            return cc
        lax.fori_loop(0, EK // 16, srow, 0)

    def process(b, g, prefetch_other, g_other):
        @pl.when(prefetch_other)
        def _():
            load_src(1 - b, g_other)
            issue_gather(1 - b)
        @pl.when(g >= 2)
        def _():
            wait_scatter(b)
        pltpu.sync_copy(dst_hbm.at[pl.ds((cbase + g) * EK, EK)], dst_v[b])
        pltpu.sync_copy(w_hbm.at[pl.ds((cbase + g) * EK, EK)], w_v[b])
        wait_gather(b)
        scale(b)
        issue_scatter(b)

    with jax.named_scope("spmm_edges"):
        load_src(0, 0)
        issue_gather(0)

        def pair(i, c):
            g0 = 2 * i
            process(0, g0, True, g0 + 1)
            process(1, g0 + 1, g0 + 2 < nch, g0 + 2)
            return c
        lax.fori_loop(0, nch // 2, pair, 0)
        process(0, nch - 1, False, 0)
        wait_scatter(1)
        wait_scatter(0)
        plsc.subcore_barrier()

    with jax.named_scope("spmm_writeout"):
        for j in range(RCH):
            r0 = sid * RPT + j * EK
            pltpu.sync_copy(acc.at[pl.ds(r0, EK)], rows_v[0])
            pltpu.sync_copy(rows_v[0], s_out.at[cid, pl.ds(r0, EK)])


_SC_MESH = plsc.VectorSubcoreMesh(core_axis_name="c", subcore_axis_name="s")

_deg_call = pl.kernel(
    _deg_body,
    out_type=jax.ShapeDtypeStruct((NC, NP), jnp.float32),
    mesh=_SC_MESH,
    scratch_types=[
        (pltpu.VMEM((EK,), jnp.int32), pltpu.VMEM((EK,), jnp.int32)),
        (pltpu.VMEM((EK,), jnp.float32), pltpu.VMEM((EK,), jnp.float32)),
        pltpu.VMEM((RPT,), jnp.float32),
        pltpu.VMEM_SHARED((NP,), jnp.float32),
        (pltpu.SemaphoreType.DMA, pltpu.SemaphoreType.DMA),
    ],
)

def _make_spmm(nch0):
  return pl.kernel(
    functools.partial(_spmm_body, nch0),
    out_type=jax.ShapeDtypeStruct((NC, NP, D), jnp.float32),
    mesh=_SC_MESH,
    scratch_types=[
        (pltpu.VMEM((EK,), jnp.int32), pltpu.VMEM((EK,), jnp.int32)),
        (pltpu.VMEM((EK,), jnp.int32), pltpu.VMEM((EK,), jnp.int32)),
        (pltpu.VMEM((EK,), jnp.float32), pltpu.VMEM((EK,), jnp.float32)),
        (pltpu.VMEM((EK, D), jnp.float32), pltpu.VMEM((EK, D), jnp.float32)),
        (pltpu.SemaphoreType.DMA, pltpu.SemaphoreType.DMA),
        (pltpu.SemaphoreType.DMA, pltpu.SemaphoreType.DMA),
        pltpu.VMEM_SHARED((NP, D), jnp.float32),
    ],
  )


_spmm_call_l1 = _make_spmm(113)
_spmm_call_l2 = _make_spmm(105)


def _tc1_body(x_ref, w1_ref, deg_ref, h1p_ref, dis_ref):
    degs = deg_ref[0] + deg_ref[1] + 1.0
    s = jnp.where(degs > 0, lax.rsqrt(jnp.where(degs > 0, degs, 1.0)), 0.0)
    h = jnp.dot(x_ref[...], w1_ref[...], preferred_element_type=jnp.float32)
    h1p_ref[...] = h * s
    dis_ref[...] = s


def _tc2_body(s1_ref, h1p_ref, dis_ref, b1_ref, w2_ref, h2p_ref):
    s = dis_ref[...]
    pre = (s1_ref[0] + s1_ref[1] + h1p_ref[...]) * s + b1_ref[...]
    o = jnp.maximum(pre, 0.0)
    h2 = jnp.dot(o, w2_ref[...], preferred_element_type=jnp.float32)
    h2p_ref[...] = h2 * s


def _tc3_body(s2_ref, h2p_ref, dis_ref, b2_ref, batch_ref, out_ref):
    i = pl.program_id(0)
    s = dis_ref[...]
    pre = (s2_ref[0] + s2_ref[1] + h2p_ref[...]) * s + b2_ref[...]
    o = jnp.maximum(pre, 0.0)
    seg = lax.broadcasted_iota(jnp.int32, (BN, G), 1)
    oh = (batch_ref[...] == seg).astype(jnp.float32)
    contrib = lax.dot_general(oh, o, (((0,), (0,)), ((), ())),
                              preferred_element_type=jnp.float32)

    @pl.when(i == 0)
    def _init():
        out_ref[...] = contrib

    @pl.when(i > 0)
    def _acc():
        out_ref[...] += contrib


_tc1_call = pl.pallas_call(
    _tc1_body,
    grid=(NB,),
    in_specs=[
        pl.BlockSpec((BN, D), lambda i: (i, 0)),
        pl.BlockSpec((D, D), lambda i: (0, 0)),
        pl.BlockSpec((NC, BN, 1), lambda i: (0, i, 0)),
    ],
    out_specs=[
        pl.BlockSpec((BN, D), lambda i: (i, 0)),
        pl.BlockSpec((BN, 1), lambda i: (i, 0)),
    ],
    out_shape=[
        jax.ShapeDtypeStruct((NP, D), jnp.float32),
        jax.ShapeDtypeStruct((NP, 1), jnp.float32),
    ],
)

_tc2_call = pl.pallas_call(
    _tc2_body,
    grid=(NB,),
    in_specs=[
        pl.BlockSpec((NC, BN, D), lambda i: (0, i, 0)),
        pl.BlockSpec((BN, D), lambda i: (i, 0)),
        pl.BlockSpec((BN, 1), lambda i: (i, 0)),
        pl.BlockSpec((1, D), lambda i: (0, 0)),
        pl.BlockSpec((D, D), lambda i: (0, 0)),
    ],
    out_specs=pl.BlockSpec((BN, D), lambda i: (i, 0)),
    out_shape=jax.ShapeDtypeStruct((NP, D), jnp.float32),
)

_tc3_call = pl.pallas_call(
    _tc3_body,
    grid=(NB,),
    in_specs=[
        pl.BlockSpec((NC, BN, D), lambda i: (0, i, 0)),
        pl.BlockSpec((BN, D), lambda i: (i, 0)),
        pl.BlockSpec((BN, 1), lambda i: (i, 0)),
        pl.BlockSpec((1, D), lambda i: (0, 0)),
        pl.BlockSpec((BN, 1), lambda i: (i, 0)),
    ],
    out_specs=pl.BlockSpec((G, D), lambda i: (0, 0)),
    out_shape=jax.ShapeDtypeStruct((G, D), jnp.float32),
)


@jax.jit
def kernel(x, edge_index, edge_weight, batch, W1, b1, W2, b2):
    e = edge_weight.shape[0]
    ept = -(-e // (NW * EK)) * EK
    epad = ept * NW
    pe = epad - e

    src = edge_index[0].astype(jnp.int32)
    dst = edge_index[1].astype(jnp.int32)
    src_p = jnp.concatenate([src, jnp.zeros((pe,), jnp.int32)])
    dst_p = jnp.concatenate([dst, jnp.zeros((pe,), jnp.int32)])
    w_p = jnp.concatenate([edge_weight, jnp.zeros((pe,), edge_weight.dtype)])

    x_p = jnp.concatenate([x, jnp.zeros((NP - N, D), x.dtype)])
    batch_p = jnp.concatenate(
        [batch.astype(jnp.int32), jnp.full((NP - N,), G, jnp.int32)]
    ).reshape(NP, 1)

    deg2 = _deg_call(dst_p, w_p).reshape(NC, NP, 1)
    h1p, dis = _tc1_call(x_p, W1, deg2)
    s1 = _spmm_call_l1(h1p, src_p, dst_p, w_p)
    h2p = _tc2_call(s1, h1p, dis, b1.reshape(1, D), W2)
    s2 = _spmm_call_l2(h2p, src_p, dst_p, w_p)
    return _tc3_call(s2, h2p, dis, b2.reshape(1, D), batch_p)
